# baseline (speedup 1.0000x reference)
import functools

import jax
import jax.numpy as jnp
from jax.experimental import pallas as pl
from jax.experimental.pallas import tpu as pltpu


def _fused_route_kernel(x0_ref, h0_ref, wq_ref, wk_ref, slot_ref, out_ref,
                        slotw_ref, *, scale, n_chunks):
    b = pl.program_id(0)

    @pl.when(b == 0)
    def _init_slot_w():
        sl = slot_ref[...]
        sl_max = jnp.max(sl, axis=-1, keepdims=True)
        se = jnp.exp(sl - sl_max)
        slotw_ref[...] = se / jnp.sum(se, axis=-1, keepdims=True)

    h0f = h0_ref[0]
    h0 = h0f.astype(jnp.bfloat16)
    x0 = x0_ref[0].astype(jnp.bfloat16)
    wq = wq_ref[...].astype(jnp.bfloat16)
    wk = wk_ref[...].astype(jnp.bfloat16)

    k = jax.lax.dot_general(x0, wk, (((1,), (1,)), ((), ())),
                            preferred_element_type=jnp.float32)
    kb = k.astype(jnp.bfloat16)

    n = h0.shape[0]
    chunk = n // n_chunks
    score_parts = []
    for c in range(n_chunks):
        h0c = h0[c * chunk:(c + 1) * chunk]
        qc = jax.lax.dot_general(h0c, wq, (((1,), (1,)), ((), ())),
                                 preferred_element_type=jnp.float32)
        sc = jax.lax.dot_general(qc.astype(jnp.bfloat16), kb,
                                 (((1,), (1,)), ((), ())),
                                 preferred_element_type=jnp.float32) * scale
        mc = jnp.max(sc)
        ec = jnp.exp(sc - mc)
        score_parts.append(jnp.log(jnp.sum(ec, axis=-1)) + mc)
    scores = jnp.concatenate(score_parts)

    smax = jnp.max(scores)
    e = jnp.exp(scores - smax)
    w_keep = e / jnp.sum(e)

    w = slotw_ref[...] * w_keep[None, :]
    z = jnp.sum(w, axis=-1, keepdims=True) + 1e-6
    out = jax.lax.dot_general(w.astype(jnp.bfloat16), h0,
                              (((1,), (0,)), ((), ())),
                              preferred_element_type=jnp.float32)
    out_ref[0] = out / z


def kernel(X0_patches, H0_patches, Wq, Wk, slot_logits):
    B, N, D = X0_patches.shape
    QK = Wq.shape[0]
    M = slot_logits.shape[0]
    scale = QK ** (-0.5)

    return pl.pallas_call(
        functools.partial(_fused_route_kernel, scale=scale, n_chunks=4),
        grid=(B,),
        in_specs=[
            pl.BlockSpec((1, N, D), lambda b: (b, 0, 0)),
            pl.BlockSpec((1, N, D), lambda b: (b, 0, 0)),
            pl.BlockSpec((QK, D), lambda b: (0, 0)),
            pl.BlockSpec((QK, D), lambda b: (0, 0)),
            pl.BlockSpec((M, N), lambda b: (0, 0)),
        ],
        out_specs=pl.BlockSpec((1, M, D), lambda b: (b, 0, 0)),
        out_shape=jax.ShapeDtypeStruct((B, M, D), jnp.float32),
        scratch_shapes=[pltpu.VMEM((M, N), jnp.float32)],
    )(X0_patches, H0_patches, Wq, Wk, slot_logits)


# speedup vs baseline: 1.2076x; 1.2076x over previous
import functools

import jax
import jax.numpy as jnp
from jax.experimental import pallas as pl
from jax.experimental.pallas import tpu as pltpu


def _fused_route_kernel(x0_ref, h0_ref, wq_ref, wk_ref, slot_ref, out_ref,
                        slotw_ref, *, scale):
    b = pl.program_id(0)

    @pl.when(b == 0)
    def _init_slot_w():
        sl = slot_ref[...]
        sl_max = jnp.max(sl, axis=-1, keepdims=True)
        se = jnp.exp(sl - sl_max)
        slotw_ref[...] = se / jnp.sum(se, axis=-1, keepdims=True)

    h0f = h0_ref[0]
    h0 = h0f.astype(jnp.bfloat16)
    x0 = x0_ref[0].astype(jnp.bfloat16)
    wq = wq_ref[...].astype(jnp.bfloat16)
    wk = wk_ref[...].astype(jnp.bfloat16)

    k = jax.lax.dot_general(x0, wk, (((1,), (1,)), ((), ())),
                            preferred_element_type=jnp.float32)
    kb = k.astype(jnp.bfloat16)

    q = jax.lax.dot_general(h0, wq, (((1,), (1,)), ((), ())),
                            preferred_element_type=jnp.float32)
    s = jax.lax.dot_general(q.astype(jnp.bfloat16), kb,
                            (((1,), (1,)), ((), ())),
                            preferred_element_type=jnp.float32) * scale
    row_max = jnp.max(s, axis=-1, keepdims=True)
    scores = jnp.log(jnp.sum(jnp.exp(s - row_max), axis=-1)) + row_max[:, 0]

    smax = jnp.max(scores)
    e = jnp.exp(scores - smax)
    w_keep = e / jnp.sum(e)

    w = slotw_ref[...] * w_keep[None, :]
    z = jnp.sum(w, axis=-1, keepdims=True) + 1e-6
    out = jax.lax.dot_general(w.astype(jnp.bfloat16), h0,
                              (((1,), (0,)), ((), ())),
                              preferred_element_type=jnp.float32)
    out_ref[0] = out / z


def kernel(X0_patches, H0_patches, Wq, Wk, slot_logits):
    B, N, D = X0_patches.shape
    QK = Wq.shape[0]
    M = slot_logits.shape[0]
    scale = QK ** (-0.5)

    return pl.pallas_call(
        functools.partial(_fused_route_kernel, scale=scale),
        grid=(B,),
        in_specs=[
            pl.BlockSpec((1, N, D), lambda b: (b, 0, 0)),
            pl.BlockSpec((1, N, D), lambda b: (b, 0, 0)),
            pl.BlockSpec((QK, D), lambda b: (0, 0)),
            pl.BlockSpec((QK, D), lambda b: (0, 0)),
            pl.BlockSpec((M, N), lambda b: (0, 0)),
        ],
        out_specs=pl.BlockSpec((1, M, D), lambda b: (b, 0, 0)),
        out_shape=jax.ShapeDtypeStruct((B, M, D), jnp.float32),
        scratch_shapes=[pltpu.VMEM((M, N), jnp.float32)],
    )(X0_patches, H0_patches, Wq, Wk, slot_logits)


# speedup vs baseline: 1.3390x; 1.1089x over previous
import functools

import jax
import jax.numpy as jnp
from jax.experimental import pallas as pl
from jax.experimental.pallas import tpu as pltpu


def _route_one_batch(x0a, x0b, h0a, h0b, wq, wk, slotwa, slotwb, scale, half):
    def _proj(xa, xb, wt):
        pa = jax.lax.dot_general(xa, wt, (((1,), (1,)), ((), ())),
                                 preferred_element_type=jnp.float32)
        pb = jax.lax.dot_general(xb, wt, (((1,), (1,)), ((), ())),
                                 preferred_element_type=jnp.float32)
        return jnp.concatenate([pa.astype(jnp.bfloat16),
                                pb.astype(jnp.bfloat16)], axis=0)

    q = _proj(h0a, h0b, wq)
    k = _proj(x0a, x0b, wk)

    st = jax.lax.dot_general(k, q, (((1,), (1,)), ((), ())),
                             preferred_element_type=jnp.float32) * scale

    cmax = jnp.max(st, axis=0, keepdims=True)
    e = jnp.exp(st - cmax)
    scores = jnp.log(jnp.sum(e, axis=0, keepdims=True)) + cmax

    smax = jnp.max(scores)
    ew = jnp.exp(scores - smax)
    w_keep = ew / jnp.sum(ew)

    w_a = slotwa * w_keep[:, :half]
    w_b = slotwb * w_keep[:, half:]
    z = (jnp.sum(w_a, axis=-1, keepdims=True)
         + jnp.sum(w_b, axis=-1, keepdims=True) + 1e-6)
    out = (jax.lax.dot_general(w_a.astype(jnp.bfloat16), h0a,
                               (((1,), (0,)), ((), ())),
                               preferred_element_type=jnp.float32)
           + jax.lax.dot_general(w_b.astype(jnp.bfloat16), h0b,
                                 (((1,), (0,)), ((), ())),
                                 preferred_element_type=jnp.float32))
    return out / z


def _fused_route_kernel(x0a_ref, x0b_ref, h0a_ref, h0b_ref, wq_ref, wk_ref,
                        slot_ref, out_ref, slotwa_ref, slotwb_ref, *, scale):
    b = pl.program_id(0)
    half = x0a_ref.shape[1]

    @pl.when(b == 0)
    def _init_slot_w():
        sl = slot_ref[...]
        sl_max = jnp.max(sl, axis=-1, keepdims=True)
        se = jnp.exp(sl - sl_max)
        slot_w = se / jnp.sum(se, axis=-1, keepdims=True)
        slotwa_ref[...] = slot_w[:, :half]
        slotwb_ref[...] = slot_w[:, half:]

    wq = wq_ref[...].astype(jnp.bfloat16)
    wk = wk_ref[...].astype(jnp.bfloat16)
    slotwa = slotwa_ref[...]
    slotwb = slotwb_ref[...]

    for i in range(2):
        out_ref[i] = _route_one_batch(
            x0a_ref[i].astype(jnp.bfloat16), x0b_ref[i].astype(jnp.bfloat16),
            h0a_ref[i].astype(jnp.bfloat16), h0b_ref[i].astype(jnp.bfloat16),
            wq, wk, slotwa, slotwb, scale, half)


def kernel(X0_patches, H0_patches, Wq, Wk, slot_logits):
    B, N, D = X0_patches.shape
    QK = Wq.shape[0]
    M = slot_logits.shape[0]
    scale = QK ** (-0.5)
    half = N // 2

    return pl.pallas_call(
        functools.partial(_fused_route_kernel, scale=scale),
        grid=(B // 2,),
        in_specs=[
            pl.BlockSpec((2, half, D), lambda b: (b, 0, 0)),
            pl.BlockSpec((2, half, D), lambda b: (b, 1, 0)),
            pl.BlockSpec((2, half, D), lambda b: (b, 0, 0)),
            pl.BlockSpec((2, half, D), lambda b: (b, 1, 0)),
            pl.BlockSpec((QK, D), lambda b: (0, 0)),
            pl.BlockSpec((QK, D), lambda b: (0, 0)),
            pl.BlockSpec((M, N), lambda b: (0, 0)),
        ],
        out_specs=pl.BlockSpec((2, M, D), lambda b: (b, 0, 0)),
        out_shape=jax.ShapeDtypeStruct((B, M, D), jnp.float32),
        scratch_shapes=[pltpu.VMEM((M, half), jnp.float32),
                        pltpu.VMEM((M, half), jnp.float32)],
    )(X0_patches, X0_patches, H0_patches, H0_patches, Wq, Wk, slot_logits)


# speedup vs baseline: 1.7216x; 1.2857x over previous
import functools

import jax
import jax.numpy as jnp
from jax.experimental import pallas as pl
from jax.experimental.pallas import tpu as pltpu


def _scores_matmuls(x0a, x0b, h0a, h0b, wq, wk, scale):
    def _proj(xa, xb, wt):
        pa = jax.lax.dot_general(xa, wt, (((1,), (1,)), ((), ())),
                                 preferred_element_type=jnp.float32)
        pb = jax.lax.dot_general(xb, wt, (((1,), (1,)), ((), ())),
                                 preferred_element_type=jnp.float32)
        return jnp.concatenate([pa.astype(jnp.bfloat16),
                                pb.astype(jnp.bfloat16)], axis=0)

    q = _proj(h0a, h0b, wq)
    k = _proj(x0a, x0b, wk)

    return jax.lax.dot_general(k, q, (((1,), (1,)), ((), ())),
                               preferred_element_type=jnp.float32) * scale


def _route_tail(st, h0a, h0b, slotwa, slotwb, half):
    cmax = jnp.max(st, axis=0, keepdims=True)
    e = jnp.exp(st - cmax)
    scores = jnp.log(jnp.sum(e, axis=0, keepdims=True)) + cmax

    smax = jnp.max(scores)
    ew = jnp.exp(scores - smax)
    w_keep = ew / jnp.sum(ew)

    w_a = slotwa * w_keep[:, :half]
    w_b = slotwb * w_keep[:, half:]
    z = (jnp.sum(w_a, axis=-1, keepdims=True)
         + jnp.sum(w_b, axis=-1, keepdims=True) + 1e-6)
    out = (jax.lax.dot_general(w_a.astype(jnp.bfloat16), h0a,
                               (((1,), (0,)), ((), ())),
                               preferred_element_type=jnp.float32)
           + jax.lax.dot_general(w_b.astype(jnp.bfloat16), h0b,
                                 (((1,), (0,)), ((), ())),
                                 preferred_element_type=jnp.float32))
    return out / z


def _fused_route_kernel(x0a_ref, x0b_ref, h0a_ref, h0b_ref, wq_ref, wk_ref,
                        slot_ref, out_ref, slotwa_ref, slotwb_ref, *, scale):
    b = pl.program_id(0)
    half = x0a_ref.shape[1]

    @pl.when(b == 0)
    def _init_slot_w():
        sl = slot_ref[...]
        sl_max = jnp.max(sl, axis=-1, keepdims=True)
        se = jnp.exp(sl - sl_max)
        slot_w = se / jnp.sum(se, axis=-1, keepdims=True)
        slotwa_ref[...] = slot_w[:, :half]
        slotwb_ref[...] = slot_w[:, half:]

    wq = wq_ref[...].astype(jnp.bfloat16)
    wk = wk_ref[...].astype(jnp.bfloat16)
    slotwa = slotwa_ref[...]
    slotwb = slotwb_ref[...]

    nb = x0a_ref.shape[0]
    h0 = [(h0a_ref[i].astype(jnp.bfloat16), h0b_ref[i].astype(jnp.bfloat16))
          for i in range(nb)]
    sts = []
    for i in range(nb):
        sts.append(_scores_matmuls(x0a_ref[i].astype(jnp.bfloat16),
                                   x0b_ref[i].astype(jnp.bfloat16),
                                   h0[i][0], h0[i][1], wq, wk, scale))
        if i >= 1:
            out_ref[i - 1] = _route_tail(sts[i - 1], h0[i - 1][0],
                                         h0[i - 1][1], slotwa, slotwb, half)
    out_ref[nb - 1] = _route_tail(sts[nb - 1], h0[nb - 1][0], h0[nb - 1][1],
                                  slotwa, slotwb, half)


def kernel(X0_patches, H0_patches, Wq, Wk, slot_logits):
    B, N, D = X0_patches.shape
    QK = Wq.shape[0]
    M = slot_logits.shape[0]
    scale = QK ** (-0.5)
    half = N // 2

    return pl.pallas_call(
        functools.partial(_fused_route_kernel, scale=scale),
        grid=(B // 4,),
        in_specs=[
            pl.BlockSpec((4, half, D), lambda b: (b, 0, 0)),
            pl.BlockSpec((4, half, D), lambda b: (b, 1, 0)),
            pl.BlockSpec((4, half, D), lambda b: (b, 0, 0)),
            pl.BlockSpec((4, half, D), lambda b: (b, 1, 0)),
            pl.BlockSpec((QK, D), lambda b: (0, 0)),
            pl.BlockSpec((QK, D), lambda b: (0, 0)),
            pl.BlockSpec((M, N), lambda b: (0, 0)),
        ],
        out_specs=pl.BlockSpec((4, M, D), lambda b: (b, 0, 0)),
        out_shape=jax.ShapeDtypeStruct((B, M, D), jnp.float32),
        scratch_shapes=[pltpu.VMEM((M, half), jnp.float32),
                        pltpu.VMEM((M, half), jnp.float32)],
    )(X0_patches, X0_patches, H0_patches, H0_patches, Wq, Wk, slot_logits)


# speedup vs baseline: 1.7308x; 1.0053x over previous
import functools

import jax
import jax.numpy as jnp
from jax.experimental import pallas as pl
from jax.experimental.pallas import tpu as pltpu


def _scores_matmuls(x0a, x0b, h0a, h0b, wq, wk, scale):
    def _proj(xa, xb, wt, s):
        pa = jax.lax.dot_general(xa, wt, (((1,), (1,)), ((), ())),
                                 preferred_element_type=jnp.float32)
        pb = jax.lax.dot_general(xb, wt, (((1,), (1,)), ((), ())),
                                 preferred_element_type=jnp.float32)
        return jnp.concatenate([(pa * s).astype(jnp.bfloat16),
                                (pb * s).astype(jnp.bfloat16)], axis=0)

    q = _proj(h0a, h0b, wq, scale)
    k = _proj(x0a, x0b, wk, 1.0)

    return jax.lax.dot_general(k, q, (((1,), (1,)), ((), ())),
                               preferred_element_type=jnp.float32)


def _route_tail(st, h0a, h0b, slotwa, slotwb, half):
    cmax = jnp.max(st, axis=0, keepdims=True)
    e = jnp.exp(st - cmax)
    scores = jnp.log(jnp.sum(e, axis=0, keepdims=True)) + cmax

    smax = jnp.max(scores)
    ew = jnp.exp(scores - smax)
    w_keep = ew / jnp.sum(ew)

    w_a = slotwa * w_keep[:, :half]
    w_b = slotwb * w_keep[:, half:]
    z = (jnp.sum(w_a, axis=-1, keepdims=True)
         + jnp.sum(w_b, axis=-1, keepdims=True) + 1e-6)
    out = (jax.lax.dot_general(w_a.astype(jnp.bfloat16), h0a,
                               (((1,), (0,)), ((), ())),
                               preferred_element_type=jnp.float32)
           + jax.lax.dot_general(w_b.astype(jnp.bfloat16), h0b,
                                 (((1,), (0,)), ((), ())),
                                 preferred_element_type=jnp.float32))
    return out / z


def _fused_route_kernel(x0a_ref, x0b_ref, h0a_ref, h0b_ref, wq_ref, wk_ref,
                        slot_ref, out_ref, slotwa_ref, slotwb_ref, *, scale):
    b = pl.program_id(0)
    half = x0a_ref.shape[1]

    @pl.when(b == 0)
    def _init_slot_w():
        sl = slot_ref[...]
        sl_max = jnp.max(sl, axis=-1, keepdims=True)
        se = jnp.exp(sl - sl_max)
        slot_w = se / jnp.sum(se, axis=-1, keepdims=True)
        slotwa_ref[...] = slot_w[:, :half]
        slotwb_ref[...] = slot_w[:, half:]

    wq = wq_ref[...].astype(jnp.bfloat16)
    wk = wk_ref[...].astype(jnp.bfloat16)
    slotwa = slotwa_ref[...]
    slotwb = slotwb_ref[...]

    nb = x0a_ref.shape[0]
    h0 = [(h0a_ref[i].astype(jnp.bfloat16), h0b_ref[i].astype(jnp.bfloat16))
          for i in range(nb)]
    sts = []
    for i in range(nb):
        sts.append(_scores_matmuls(x0a_ref[i].astype(jnp.bfloat16),
                                   x0b_ref[i].astype(jnp.bfloat16),
                                   h0[i][0], h0[i][1], wq, wk, scale))
        if i >= 1:
            out_ref[i - 1] = _route_tail(sts[i - 1], h0[i - 1][0],
                                         h0[i - 1][1], slotwa, slotwb, half)
    out_ref[nb - 1] = _route_tail(sts[nb - 1], h0[nb - 1][0], h0[nb - 1][1],
                                  slotwa, slotwb, half)


def kernel(X0_patches, H0_patches, Wq, Wk, slot_logits):
    B, N, D = X0_patches.shape
    QK = Wq.shape[0]
    M = slot_logits.shape[0]
    scale = QK ** (-0.5)
    half = N // 2

    return pl.pallas_call(
        functools.partial(_fused_route_kernel, scale=scale),
        grid=(B // 4,),
        in_specs=[
            pl.BlockSpec((4, half, D), lambda b: (b, 0, 0)),
            pl.BlockSpec((4, half, D), lambda b: (b, 1, 0)),
            pl.BlockSpec((4, half, D), lambda b: (b, 0, 0)),
            pl.BlockSpec((4, half, D), lambda b: (b, 1, 0)),
            pl.BlockSpec((QK, D), lambda b: (0, 0)),
            pl.BlockSpec((QK, D), lambda b: (0, 0)),
            pl.BlockSpec((M, N), lambda b: (0, 0)),
        ],
        out_specs=pl.BlockSpec((4, M, D), lambda b: (b, 0, 0)),
        out_shape=jax.ShapeDtypeStruct((B, M, D), jnp.float32),
        scratch_shapes=[pltpu.VMEM((M, half), jnp.float32),
                        pltpu.VMEM((M, half), jnp.float32)],
    )(X0_patches, X0_patches, H0_patches, H0_patches, Wq, Wk, slot_logits)


# speedup vs baseline: 1.7333x; 1.0014x over previous
import functools

import jax
import jax.numpy as jnp
from jax.experimental import pallas as pl
from jax.experimental.pallas import tpu as pltpu


def _scores_matmuls(x0a, x0b, h0a, h0b, wq, wk, scale):
    def _proj(xa, xb, wt, s):
        pa = jax.lax.dot_general(xa, wt, (((1,), (1,)), ((), ())),
                                 preferred_element_type=jnp.float32)
        pb = jax.lax.dot_general(xb, wt, (((1,), (1,)), ((), ())),
                                 preferred_element_type=jnp.float32)
        return jnp.concatenate([(pa * s).astype(jnp.bfloat16),
                                (pb * s).astype(jnp.bfloat16)], axis=0)

    q = _proj(h0a, h0b, wq, scale)
    k = _proj(x0a, x0b, wk, 1.0)

    return jax.lax.dot_general(k, q, (((1,), (1,)), ((), ())),
                               preferred_element_type=jnp.float32)


def _route_tail(st, h0a, h0b, slotwa, slotwb, half):
    cmax = jnp.max(st, axis=0, keepdims=True)
    e = jnp.exp(st - cmax)
    scores = jnp.log(jnp.sum(e, axis=0, keepdims=True)) + cmax

    smax = jnp.max(scores)
    ew = jnp.exp(scores - smax)
    w_keep = ew / jnp.sum(ew)

    w_a = slotwa * w_keep[:, :half]
    w_b = slotwb * w_keep[:, half:]
    z = (jnp.sum(w_a, axis=-1, keepdims=True)
         + jnp.sum(w_b, axis=-1, keepdims=True) + 1e-6)
    out = (jax.lax.dot_general(w_a, h0a, (((1,), (0,)), ((), ())),
                               preferred_element_type=jnp.float32)
           + jax.lax.dot_general(w_b, h0b, (((1,), (0,)), ((), ())),
                                 preferred_element_type=jnp.float32))
    return out / z


def _fused_route_kernel(x0a_ref, x0b_ref, h0a_ref, h0b_ref, wq_ref, wk_ref,
                        slot_ref, out_ref, slotwa_ref, slotwb_ref, *, scale):
    b = pl.program_id(0)
    half = x0a_ref.shape[1]

    @pl.when(b == 0)
    def _init_slot_w():
        sl = slot_ref[...]
        sl_max = jnp.max(sl, axis=-1, keepdims=True)
        se = jnp.exp(sl - sl_max)
        slot_w = se / jnp.sum(se, axis=-1, keepdims=True)
        slotwa_ref[...] = slot_w[:, :half]
        slotwb_ref[...] = slot_w[:, half:]

    wq = wq_ref[...]
    wk = wk_ref[...]
    slotwa = slotwa_ref[...]
    slotwb = slotwb_ref[...]

    nb = x0a_ref.shape[0]
    h0 = [(h0a_ref[i], h0b_ref[i]) for i in range(nb)]
    sts = []
    for i in range(nb):
        sts.append(_scores_matmuls(x0a_ref[i], x0b_ref[i],
                                   h0[i][0], h0[i][1], wq, wk, scale))
        if i >= 1:
            out_ref[i - 1] = _route_tail(sts[i - 1], h0[i - 1][0],
                                         h0[i - 1][1], slotwa, slotwb, half)
    out_ref[nb - 1] = _route_tail(sts[nb - 1], h0[nb - 1][0], h0[nb - 1][1],
                                  slotwa, slotwb, half)


def kernel(X0_patches, H0_patches, Wq, Wk, slot_logits):
    B, N, D = X0_patches.shape
    QK = Wq.shape[0]
    M = slot_logits.shape[0]
    scale = QK ** (-0.5)
    half = N // 2

    return pl.pallas_call(
        functools.partial(_fused_route_kernel, scale=scale),
        grid=(B // 4,),
        in_specs=[
            pl.BlockSpec((4, half, D), lambda b: (b, 0, 0)),
            pl.BlockSpec((4, half, D), lambda b: (b, 1, 0)),
            pl.BlockSpec((4, half, D), lambda b: (b, 0, 0)),
            pl.BlockSpec((4, half, D), lambda b: (b, 1, 0)),
            pl.BlockSpec((QK, D), lambda b: (0, 0)),
            pl.BlockSpec((QK, D), lambda b: (0, 0)),
            pl.BlockSpec((M, N), lambda b: (0, 0)),
        ],
        out_specs=pl.BlockSpec((4, M, D), lambda b: (b, 0, 0)),
        out_shape=jax.ShapeDtypeStruct((B, M, D), jnp.float32),
        scratch_shapes=[pltpu.VMEM((M, half), jnp.float32),
                        pltpu.VMEM((M, half), jnp.float32)],
    )(X0_patches, X0_patches, H0_patches, H0_patches, Wq, Wk, slot_logits)
